# baseline (speedup 1.0000x reference)
import jax
import jax.numpy as jnp
from jax import lax
from jax.experimental import pallas as pl
from jax.experimental.pallas import tpu as pltpu
from jax.experimental.pallas import tpu_sc as plsc

N = 10000
E = 320000
D = 128

NC = 2
NS = 16
L = 16

N_PAD = 10240
ROWS_PER_TILE = N_PAD // NS
E_PAD = 327680
EROWS = E_PAD // 128
AROWS = EROWS // NS
BROWS = EROWS // (NC * NS)


def _rsqrt16(x):
    i = lax.bitcast_convert_type(x, jnp.int32)
    i = jnp.int32(0x5F3759DF) - lax.shift_right_logical(i, 1)
    y = lax.bitcast_convert_type(i, jnp.float32)
    for _ in range(3):
        y = y * (1.5 - 0.5 * x * y * y)
    return y


NGRP = BROWS // 8


def _sc_body(feat_hbm, src_hbm, dst_hbm, w_hbm, part_hbm, ideg_hbm,
             agg_sh, odeg_sh, ideg_sh, onorm_sh,
             src_g, dst_g, w_g, nrm_g, rows3,
             sem_idx, sem_g, sem_n, sem_s):
    c = lax.axis_index("c")
    s = lax.axis_index("s")
    r0 = s * ROWS_PER_TILE

    zeros16 = jnp.zeros((L,), jnp.float32)
    ones16 = jnp.ones((L,), jnp.float32)

    def wait_gather():
        pltpu.make_async_copy(feat_hbm.at[pl.ds(0, 128)], rows3.at[0],
                              sem_g).wait()

    def wait_scatter():
        pltpu.make_async_copy(rows3.at[0], agg_sh.at[pl.ds(0, 128)],
                              sem_s).wait()

    def wait_norm():
        pltpu.make_async_copy(onorm_sh.at[pl.ds(0, 128)], nrm_g.at[0, 0],
                              sem_n).wait()

    def wait_ones():
        pltpu.make_async_copy(w_g.at[0, 0], odeg_sh.at[pl.ds(0, 128)],
                              sem_s).wait()

    def wait_group_load():
        pltpu.make_async_copy(src_hbm.at[pl.ds(0, 8)], src_g.at[0],
                              sem_idx).wait()

    def zero_row(i, _):
        for k in range(D // L):
            rows3[0, i, pl.ds(k * L, L)] = zeros16
        return 0

    lax.fori_loop(0, 128, zero_row, 0)
    for k in range(128 // L):
        w_g[0, 0, pl.ds(k * L, L)] = ones16
    for q in range(ROWS_PER_TILE // 128):
        pltpu.sync_copy(rows3.at[0, 0], odeg_sh.at[pl.ds(r0 + q * 128, 128)])
        pltpu.sync_copy(rows3.at[0, 0], ideg_sh.at[pl.ds(r0 + q * 128, 128)])
        pltpu.sync_copy(rows3.at[0], agg_sh.at[pl.ds(r0 + q * 128, 128)])
    plsc.subcore_barrier()

    ones_row = w_g.at[0, 0]
    baseA = s * AROWS

    def count_degrees(edges_hbm, deg_target):
        pltpu.sync_copy(edges_hbm.at[pl.ds(baseA, 8)], src_g.at[0])
        pltpu.async_copy(edges_hbm.at[pl.ds(baseA + 8, 8)], src_g.at[1],
                         sem_idx)

        def grp(gi, _):
            p = gi % 2
            for jj in range(8):
                pltpu.async_copy(ones_row, deg_target.at[src_g.at[p, jj]],
                                 sem_s, add=True)

            @pl.when(gi < AROWS // 8 - 1)
            def _():
                wait_group_load()

            for jj in range(8):
                wait_ones()

            @pl.when(gi < AROWS // 8 - 2)
            def _():
                pltpu.async_copy(
                    edges_hbm.at[pl.ds(baseA + (gi + 2) * 8, 8)],
                    src_g.at[p], sem_idx)
            return 0

        lax.fori_loop(0, AROWS // 8, grp, 0)

    count_degrees(src_hbm, odeg_sh)
    count_degrees(dst_hbm, ideg_sh)
    plsc.subcore_barrier()

    stage = nrm_g.at[0, 0]
    for q in range(ROWS_PER_TILE // 128):
        pltpu.sync_copy(odeg_sh.at[pl.ds(r0 + q * 128, 128)], stage)
        for k in range(128 // L):
            x = jnp.maximum(nrm_g[0, 0, pl.ds(k * L, L)], 1.0)
            nrm_g[0, 0, pl.ds(k * L, L)] = _rsqrt16(x)
        pltpu.sync_copy(stage, onorm_sh.at[pl.ds(r0 + q * 128, 128)])

    @pl.when(c == 0)
    def _():
        pltpu.sync_copy(ideg_sh.at[pl.ds(r0, ROWS_PER_TILE)],
                        ideg_hbm.at[pl.ds(r0, ROWS_PER_TILE)])

    plsc.subcore_barrier()

    baseB = (c * NS + s) * BROWS

    def load_grp(g, slot):
        pltpu.async_copy(src_hbm.at[pl.ds(baseB + g * 8, 8)],
                         src_g.at[slot], sem_idx)
        pltpu.async_copy(dst_hbm.at[pl.ds(baseB + g * 8, 8)],
                         dst_g.at[slot], sem_idx)
        pltpu.async_copy(w_hbm.at[pl.ds(baseB + g * 8, 8)],
                         w_g.at[slot], sem_idx)

    def norm_gathers(slot):
        for jj in range(8):
            pltpu.async_copy(onorm_sh.at[src_g.at[slot, jj]],
                             nrm_g.at[slot, jj], sem_n)

    load_grp(0, 0)
    for _ in range(3):
        wait_group_load()
    norm_gathers(0)
    pltpu.async_copy(feat_hbm.at[src_g.at[0, 0]], rows3.at[0], sem_g)

    def chunk(j, _):
        g = j // 8
        jj = j % 8
        p = g % 2
        b = j % 2

        wait_gather()

        @pl.when(jj == 0)
        def _():
            for _ in range(8):
                wait_norm()

        def scale_body(gg, _):
            wn16 = (w_g[p, jj, pl.ds(gg * L, L)]
                    * nrm_g[p, jj, pl.ds(gg * L, L)])
            for quad in range(L // 4):
                vals = []
                for r in range(4):
                    i = gg * L + quad * 4 + r
                    sc = wn16[quad * 4 + r]
                    vals.extend(
                        rows3[b, i, pl.ds(k * L, L)] * sc
                        for k in range(D // L))
                for r in range(4):
                    i = gg * L + quad * 4 + r
                    for k in range(D // L):
                        rows3[b, i, pl.ds(k * L, L)] = vals[r * (D // L) + k]
            return 0

        def scale_range(lo, hi):
            @plsc.parallel_loop(lo, hi, 1, unroll=2)
            def _(gg):
                scale_body(gg, 0)

        scale_range(0, 4)

        @pl.when(j > 0)
        def _():
            wait_scatter()

        @pl.when(jnp.logical_and(jj == 0, g + 1 < NGRP))
        def _():
            load_grp(g + 1, 1 - p)

        @pl.when(jnp.logical_and(jj == 7, j < BROWS - 1))
        def _():
            for _ in range(3):
                wait_group_load()
            norm_gathers(1 - p)

        @pl.when(j < BROWS - 1)
        def _():
            g1 = (j + 1) // 8
            pltpu.async_copy(
                feat_hbm.at[src_g.at[g1 % 2, (j + 1) % 8]],
                rows3.at[1 - b], sem_g)

        scale_range(4, 8)
        pltpu.async_copy(rows3.at[b], agg_sh.at[dst_g.at[p, jj]], sem_s,
                         add=True)
        return 0

    lax.fori_loop(0, BROWS, chunk, 0)
    wait_scatter()
    plsc.subcore_barrier()

    pltpu.sync_copy(agg_sh.at[pl.ds(r0, ROWS_PER_TILE)],
                    part_hbm.at[c, pl.ds(r0, ROWS_PER_TILE)])


def _sc_aggregate(feat_p, src2, dst2, w2):
    mesh = plsc.VectorSubcoreMesh(core_axis_name="c", subcore_axis_name="s",
                                  num_cores=NC, num_subcores=NS)
    return pl.kernel(
        _sc_body,
        out_type=(
            jax.ShapeDtypeStruct((NC, N_PAD, D), jnp.float32),
            jax.ShapeDtypeStruct((N_PAD,), jnp.float32),
        ),
        mesh=mesh,
        compiler_params=pltpu.CompilerParams(use_tc_tiling_on_sc=False),
        scratch_types=[
            pltpu.VMEM_SHARED((N_PAD, D), jnp.float32),
            pltpu.VMEM_SHARED((N_PAD,), jnp.float32),
            pltpu.VMEM_SHARED((N_PAD,), jnp.float32),
            pltpu.VMEM_SHARED((N_PAD,), jnp.float32),
            pltpu.VMEM((2, 8, 128), jnp.int32),
            pltpu.VMEM((2, 8, 128), jnp.int32),
            pltpu.VMEM((2, 8, 128), jnp.float32),
            pltpu.VMEM((2, 8, 128), jnp.float32),
            pltpu.VMEM((2, 128, D), jnp.float32),
            pltpu.SemaphoreType.DMA,
            pltpu.SemaphoreType.DMA,
            pltpu.SemaphoreType.DMA,
            pltpu.SemaphoreType.DMA,
        ],
        name="gcn_sc_aggregate",
    )(feat_p, src2, dst2, w2)


ROWS_BLK = 256
N_BLKS = N_PAD // ROWS_BLK


def _tc_tail_body(pa_ref, ideg_ref, b3_ref, w1_ref, bd1_ref, w2_ref, bd2_ref,
                  w3_ref, bd3_ref, out_ref, acc_ref):
    i = pl.program_id(0)

    @pl.when(i == 0)
    def _():
        acc_ref[...] = jnp.zeros_like(acc_ref)

    agg = pa_ref[0] + pa_ref[1]
    ideg = jnp.maximum(ideg_ref[...], 1.0)
    inorm = lax.rsqrt(ideg)
    h = jnp.maximum(agg * inorm + b3_ref[...], 0.0)
    row = i * ROWS_BLK + lax.broadcasted_iota(jnp.int32, (ROWS_BLK, 1), 0)
    h = jnp.where(row < N, h, 0.0)
    acc_ref[...] += jnp.sum(h, axis=0, keepdims=True)

    @pl.when(i == N_BLKS - 1)
    def _():
        hg = acc_ref[...] * (1.0 / N)
        m = jnp.maximum(jnp.dot(hg, w1_ref[...],
                                preferred_element_type=jnp.float32)
                        + bd1_ref[...], 0.0)
        m = jnp.maximum(jnp.dot(m, w2_ref[...],
                                preferred_element_type=jnp.float32)
                        + bd2_ref[...], 0.0)
        z = jnp.dot(m, w3_ref[...],
                    preferred_element_type=jnp.float32) + bd3_ref[...]
        out_ref[...] = 1.0 / (1.0 + jnp.exp(-z))


def _tc_tail(part, ideg2, b3, W1, bd1, W2, bd2, W3, bd3):
    full = lambda shape: pl.BlockSpec(shape, lambda i: tuple(0 for _ in shape))
    return pl.pallas_call(
        _tc_tail_body,
        grid=(N_BLKS,),
        in_specs=[
            pl.BlockSpec((NC, ROWS_BLK, D), lambda i: (0, i, 0)),
            pl.BlockSpec((ROWS_BLK, 1), lambda i: (i, 0)),
            full((1, D)),
            full((D, 16)), full((1, 16)),
            full((16, 8)), full((1, 8)),
            full((8, 1)), full((1, 1)),
        ],
        out_specs=pl.BlockSpec((1, 1), lambda i: (0, 0)),
        out_shape=jax.ShapeDtypeStruct((1, 1), jnp.float32),
        scratch_shapes=[pltpu.VMEM((1, D), jnp.float32)],
    )(part, ideg2, b3, W1, bd1, W2, bd2, W3, bd3)


def kernel(features, edge_index, e_weight, b1, b2, b3, W1, bd1, W2, bd2, W3,
           bd3):
    del b1, b2
    src = edge_index[0].astype(jnp.int32)
    dst = edge_index[1].astype(jnp.int32)
    w = e_weight.astype(jnp.float32)

    npad = E_PAD - E
    pad_idx = (N + (jnp.arange(npad, dtype=jnp.int32) % (N_PAD - N)))
    src2 = jnp.concatenate([src, pad_idx]).reshape(EROWS, 128)
    dst2 = jnp.concatenate([dst, pad_idx]).reshape(EROWS, 128)
    w2 = jnp.concatenate([w, jnp.zeros((npad,), jnp.float32)]).reshape(
        EROWS, 128)
    feat_p = jnp.pad(features, ((0, N_PAD - N), (0, 0)))

    part, ideg = _sc_aggregate(feat_p, src2, dst2, w2)
    return _tc_tail(part, ideg.reshape(N_PAD, 1), b3.reshape(1, D),
                    W1, bd1.reshape(1, 16), W2, bd2.reshape(1, 8),
                    W3, bd3.reshape(1, 1))


# speedup vs baseline: 1.0037x; 1.0037x over previous
import jax
import jax.numpy as jnp
from jax import lax
from jax.experimental import pallas as pl
from jax.experimental.pallas import tpu as pltpu
from jax.experimental.pallas import tpu_sc as plsc

N = 10000
E = 320000
D = 128

NC = 2
NS = 16
L = 16

N_PAD = 10240
ROWS_PER_TILE = N_PAD // NS
E_PAD = 327680
EROWS = E_PAD // 128
AROWS = EROWS // NS
BROWS = EROWS // (NC * NS)


def _rsqrt16(x):
    i = lax.bitcast_convert_type(x, jnp.int32)
    i = jnp.int32(0x5F3759DF) - lax.shift_right_logical(i, 1)
    y = lax.bitcast_convert_type(i, jnp.float32)
    for _ in range(3):
        y = y * (1.5 - 0.5 * x * y * y)
    return y


NGRP = BROWS // 8


def _sc_body(feat_hbm, src_hbm, dst_hbm, w_hbm, part_hbm, ideg_hbm,
             agg_sh, odeg_sh, ideg_sh, onorm_sh,
             src_g, dst_g, w_g, nrm_g, rows3,
             sem_idx, sem_g, sem_n, sem_s):
    c = lax.axis_index("c")
    s = lax.axis_index("s")
    r0 = s * ROWS_PER_TILE

    zeros16 = jnp.zeros((L,), jnp.float32)
    ones16 = jnp.ones((L,), jnp.float32)

    def wait_gather():
        pltpu.make_async_copy(feat_hbm.at[pl.ds(0, 128)], rows3.at[0],
                              sem_g).wait()

    def wait_scatter():
        pltpu.make_async_copy(rows3.at[0], agg_sh.at[pl.ds(0, 128)],
                              sem_s).wait()

    def wait_norm():
        pltpu.make_async_copy(onorm_sh.at[pl.ds(0, 128)], nrm_g.at[0, 0],
                              sem_n).wait()

    def wait_ones():
        pltpu.make_async_copy(w_g.at[0, 0], odeg_sh.at[pl.ds(0, 128)],
                              sem_s).wait()

    def wait_group_load():
        pltpu.make_async_copy(src_hbm.at[pl.ds(0, 8)], src_g.at[0],
                              sem_idx).wait()

    def zero_row(i, _):
        for k in range(D // L):
            rows3[0, i, pl.ds(k * L, L)] = zeros16
        return 0

    with jax.named_scope("ph_zero"):
        lax.fori_loop(0, 128, zero_row, 0)
    for k in range(128 // L):
        w_g[0, 0, pl.ds(k * L, L)] = ones16
    for q in range(ROWS_PER_TILE // 128):
        pltpu.sync_copy(rows3.at[0, 0], odeg_sh.at[pl.ds(r0 + q * 128, 128)])
        pltpu.sync_copy(rows3.at[0, 0], ideg_sh.at[pl.ds(r0 + q * 128, 128)])
        pltpu.sync_copy(rows3.at[0], agg_sh.at[pl.ds(r0 + q * 128, 128)])
    plsc.subcore_barrier()

    ones_row = w_g.at[0, 0]
    baseA = s * AROWS

    def count_degrees(edges_hbm, deg_target):
        pltpu.sync_copy(edges_hbm.at[pl.ds(baseA, 8)], src_g.at[0])
        pltpu.async_copy(edges_hbm.at[pl.ds(baseA + 8, 8)], src_g.at[1],
                         sem_idx)

        def grp(gi, _):
            p = gi % 2
            for jj in range(8):
                pltpu.async_copy(ones_row, deg_target.at[src_g.at[p, jj]],
                                 sem_s, add=True)

            @pl.when(gi < AROWS // 8 - 1)
            def _():
                wait_group_load()

            for jj in range(8):
                wait_ones()

            @pl.when(gi < AROWS // 8 - 2)
            def _():
                pltpu.async_copy(
                    edges_hbm.at[pl.ds(baseA + (gi + 2) * 8, 8)],
                    src_g.at[p], sem_idx)
            return 0

        lax.fori_loop(0, AROWS // 8, grp, 0)

    with jax.named_scope("ph_degrees"):
        count_degrees(src_hbm, odeg_sh)
        count_degrees(dst_hbm, ideg_sh)
    plsc.subcore_barrier()

    stage = nrm_g.at[0, 0]
    for q in range(ROWS_PER_TILE // 128):
        pltpu.sync_copy(odeg_sh.at[pl.ds(r0 + q * 128, 128)], stage)
        for k in range(128 // L):
            x = jnp.maximum(nrm_g[0, 0, pl.ds(k * L, L)], 1.0)
            nrm_g[0, 0, pl.ds(k * L, L)] = _rsqrt16(x)
        pltpu.sync_copy(stage, onorm_sh.at[pl.ds(r0 + q * 128, 128)])

    @pl.when(c == 0)
    def _():
        pltpu.sync_copy(ideg_sh.at[pl.ds(r0, ROWS_PER_TILE)],
                        ideg_hbm.at[pl.ds(r0, ROWS_PER_TILE)])

    plsc.subcore_barrier()

    baseB = (c * NS + s) * BROWS

    def load_grp(g, slot):
        pltpu.async_copy(src_hbm.at[pl.ds(baseB + g * 8, 8)],
                         src_g.at[slot], sem_idx)
        pltpu.async_copy(dst_hbm.at[pl.ds(baseB + g * 8, 8)],
                         dst_g.at[slot], sem_idx)
        pltpu.async_copy(w_hbm.at[pl.ds(baseB + g * 8, 8)],
                         w_g.at[slot], sem_idx)

    def norm_gathers(slot):
        for jj in range(8):
            pltpu.async_copy(onorm_sh.at[src_g.at[slot, jj]],
                             nrm_g.at[slot, jj], sem_n)

    load_grp(0, 0)
    for _ in range(3):
        wait_group_load()
    norm_gathers(0)
    pltpu.async_copy(feat_hbm.at[src_g.at[0, 0]], rows3.at[0], sem_g)

    def chunk(j, _):
        g = j // 8
        jj = j % 8
        p = g % 2
        b = j % 2

        wait_gather()

        @pl.when(jj == 0)
        def _():
            for _ in range(8):
                wait_norm()

        def scale_body(gg, _):
            wn16 = (w_g[p, jj, pl.ds(gg * L, L)]
                    * nrm_g[p, jj, pl.ds(gg * L, L)])
            for quad in range(L // 4):
                vals = []
                for r in range(4):
                    i = gg * L + quad * 4 + r
                    sc = wn16[quad * 4 + r]
                    vals.extend(
                        rows3[b, i, pl.ds(k * L, L)] * sc
                        for k in range(D // L))
                for r in range(4):
                    i = gg * L + quad * 4 + r
                    for k in range(D // L):
                        rows3[b, i, pl.ds(k * L, L)] = vals[r * (D // L) + k]
            return 0

        def scale_range(lo, hi):
            @plsc.parallel_loop(lo, hi, 1, unroll=2)
            def _(gg):
                scale_body(gg, 0)

        scale_range(0, 4)

        @pl.when(j > 0)
        def _():
            wait_scatter()

        @pl.when(jnp.logical_and(jj == 0, g + 1 < NGRP))
        def _():
            load_grp(g + 1, 1 - p)

        @pl.when(jnp.logical_and(jj == 7, j < BROWS - 1))
        def _():
            for _ in range(3):
                wait_group_load()
            norm_gathers(1 - p)

        @pl.when(j < BROWS - 1)
        def _():
            g1 = (j + 1) // 8
            pltpu.async_copy(
                feat_hbm.at[src_g.at[g1 % 2, (j + 1) % 8]],
                rows3.at[1 - b], sem_g)

        scale_range(4, 8)
        pltpu.async_copy(rows3.at[b], agg_sh.at[dst_g.at[p, jj]], sem_s,
                         add=True)
        return 0

    with jax.named_scope("ph_aggregate"):
        lax.fori_loop(0, BROWS, chunk, 0)
    wait_scatter()
    plsc.subcore_barrier()

    pltpu.sync_copy(agg_sh.at[pl.ds(r0, ROWS_PER_TILE)],
                    part_hbm.at[c, pl.ds(r0, ROWS_PER_TILE)])


def _sc_aggregate(feat_p, src2, dst2, w2):
    mesh = plsc.VectorSubcoreMesh(core_axis_name="c", subcore_axis_name="s",
                                  num_cores=NC, num_subcores=NS)
    return pl.kernel(
        _sc_body,
        out_type=(
            jax.ShapeDtypeStruct((NC, N_PAD, D), jnp.float32),
            jax.ShapeDtypeStruct((N_PAD,), jnp.float32),
        ),
        mesh=mesh,
        compiler_params=pltpu.CompilerParams(use_tc_tiling_on_sc=False),
        scratch_types=[
            pltpu.VMEM_SHARED((N_PAD, D), jnp.float32),
            pltpu.VMEM_SHARED((N_PAD,), jnp.float32),
            pltpu.VMEM_SHARED((N_PAD,), jnp.float32),
            pltpu.VMEM_SHARED((N_PAD,), jnp.float32),
            pltpu.VMEM((2, 8, 128), jnp.int32),
            pltpu.VMEM((2, 8, 128), jnp.int32),
            pltpu.VMEM((2, 8, 128), jnp.float32),
            pltpu.VMEM((2, 8, 128), jnp.float32),
            pltpu.VMEM((2, 128, D), jnp.float32),
            pltpu.SemaphoreType.DMA,
            pltpu.SemaphoreType.DMA,
            pltpu.SemaphoreType.DMA,
            pltpu.SemaphoreType.DMA,
        ],
        name="gcn_sc_aggregate",
    )(feat_p, src2, dst2, w2)


ROWS_BLK = 256
N_BLKS = N_PAD // ROWS_BLK


def _tc_tail_body(pa_ref, ideg_ref, b3_ref, w1_ref, bd1_ref, w2_ref, bd2_ref,
                  w3_ref, bd3_ref, out_ref, acc_ref):
    i = pl.program_id(0)

    @pl.when(i == 0)
    def _():
        acc_ref[...] = jnp.zeros_like(acc_ref)

    agg = pa_ref[0] + pa_ref[1]
    ideg = jnp.maximum(ideg_ref[...], 1.0)
    inorm = lax.rsqrt(ideg)
    h = jnp.maximum(agg * inorm + b3_ref[...], 0.0)
    row = i * ROWS_BLK + lax.broadcasted_iota(jnp.int32, (ROWS_BLK, 1), 0)
    h = jnp.where(row < N, h, 0.0)
    acc_ref[...] += jnp.sum(h, axis=0, keepdims=True)

    @pl.when(i == N_BLKS - 1)
    def _():
        hg = acc_ref[...] * (1.0 / N)
        m = jnp.maximum(jnp.dot(hg, w1_ref[...],
                                preferred_element_type=jnp.float32)
                        + bd1_ref[...], 0.0)
        m = jnp.maximum(jnp.dot(m, w2_ref[...],
                                preferred_element_type=jnp.float32)
                        + bd2_ref[...], 0.0)
        z = jnp.dot(m, w3_ref[...],
                    preferred_element_type=jnp.float32) + bd3_ref[...]
        out_ref[...] = 1.0 / (1.0 + jnp.exp(-z))


def _tc_tail(part, ideg2, b3, W1, bd1, W2, bd2, W3, bd3):
    full = lambda shape: pl.BlockSpec(shape, lambda i: tuple(0 for _ in shape))
    return pl.pallas_call(
        _tc_tail_body,
        grid=(N_BLKS,),
        in_specs=[
            pl.BlockSpec((NC, ROWS_BLK, D), lambda i: (0, i, 0)),
            pl.BlockSpec((ROWS_BLK, 1), lambda i: (i, 0)),
            full((1, D)),
            full((D, 16)), full((1, 16)),
            full((16, 8)), full((1, 8)),
            full((8, 1)), full((1, 1)),
        ],
        out_specs=pl.BlockSpec((1, 1), lambda i: (0, 0)),
        out_shape=jax.ShapeDtypeStruct((1, 1), jnp.float32),
        scratch_shapes=[pltpu.VMEM((1, D), jnp.float32)],
    )(part, ideg2, b3, W1, bd1, W2, bd2, W3, bd3)


def kernel(features, edge_index, e_weight, b1, b2, b3, W1, bd1, W2, bd2, W3,
           bd3):
    del b1, b2
    src = edge_index[0].astype(jnp.int32)
    dst = edge_index[1].astype(jnp.int32)
    w = e_weight.astype(jnp.float32)

    npad = E_PAD - E
    pad_idx = (N + (jnp.arange(npad, dtype=jnp.int32) % (N_PAD - N)))
    src2 = jnp.concatenate([src, pad_idx]).reshape(EROWS, 128)
    dst2 = jnp.concatenate([dst, pad_idx]).reshape(EROWS, 128)
    w2 = jnp.concatenate([w, jnp.zeros((npad,), jnp.float32)]).reshape(
        EROWS, 128)
    feat_p = jnp.pad(features, ((0, N_PAD - N), (0, 0)))

    part, ideg = _sc_aggregate(feat_p, src2, dst2, w2)
    return _tc_tail(part, ideg.reshape(N_PAD, 1), b3.reshape(1, D),
                    W1, bd1.reshape(1, 16), W2, bd2.reshape(1, 8),
                    W3, bd3.reshape(1, 1))


# speedup vs baseline: 1.1078x; 1.1037x over previous
import jax
import jax.numpy as jnp
from jax import lax
from jax.experimental import pallas as pl
from jax.experimental.pallas import tpu as pltpu
from jax.experimental.pallas import tpu_sc as plsc

N = 10000
E = 320000
D = 128

NC = 2
NS = 16
L = 16

N_PAD = 10240
ROWS_PER_TILE = N_PAD // NS
E_PAD = 327680
EROWS = E_PAD // 128
AROWS = EROWS // NS
BROWS = EROWS // (NC * NS)


def _rsqrt16(x):
    i = lax.bitcast_convert_type(x, jnp.int32)
    i = jnp.int32(0x5F3759DF) - lax.shift_right_logical(i, 1)
    y = lax.bitcast_convert_type(i, jnp.float32)
    for _ in range(3):
        y = y * (1.5 - 0.5 * x * y * y)
    return y


NGRP = BROWS // 8


def _sc_body(feat_hbm, ei_hbm, w_hbm, part_hbm, ideg_hbm,
             agg_sh, odeg_sh, ideg_sh, onorm_sh,
             src_g, dst_g, w_g, nrm_g, rows3,
             sem_idx, sem_g, sem_n, sem_s):
    src_hbm = ei_hbm.at[0]
    dst_hbm = ei_hbm.at[1]
    c = lax.axis_index("c")
    s = lax.axis_index("s")
    r0 = s * ROWS_PER_TILE

    zeros16 = jnp.zeros((L,), jnp.float32)
    ones16 = jnp.ones((L,), jnp.float32)

    def wait_gather():
        pltpu.make_async_copy(feat_hbm.at[pl.ds(0, 128)], rows3.at[0],
                              sem_g).wait()

    def wait_scatter():
        pltpu.make_async_copy(rows3.at[0], agg_sh.at[pl.ds(0, 128)],
                              sem_s).wait()

    def wait_norm():
        pltpu.make_async_copy(onorm_sh.at[pl.ds(0, 128)], nrm_g.at[0, 0],
                              sem_n).wait()

    def wait_ones():
        pltpu.make_async_copy(w_g.at[0, 0], odeg_sh.at[pl.ds(0, 128)],
                              sem_s).wait()

    def wait_group_load():
        pltpu.make_async_copy(src_hbm.at[pl.ds(0, 8)], src_g.at[0],
                              sem_idx).wait()

    def zero_row(i, _):
        for k in range(D // L):
            rows3[0, i, pl.ds(k * L, L)] = zeros16
        return 0

    with jax.named_scope("ph_zero"):
        lax.fori_loop(0, 128, zero_row, 0)
    for k in range(128 // L):
        w_g[0, 0, pl.ds(k * L, L)] = ones16
    for q in range(ROWS_PER_TILE // 128):
        pltpu.sync_copy(rows3.at[0, 0], odeg_sh.at[pl.ds(r0 + q * 128, 128)])
        pltpu.sync_copy(rows3.at[0, 0], ideg_sh.at[pl.ds(r0 + q * 128, 128)])
        pltpu.sync_copy(rows3.at[0], agg_sh.at[pl.ds(r0 + q * 128, 128)])
    plsc.subcore_barrier()

    ones_row = w_g.at[0, 0]
    baseA = s * AROWS

    def count_degrees(edges_hbm, deg_target):
        pltpu.sync_copy(edges_hbm.at[pl.ds(baseA, 8)], src_g.at[0])
        pltpu.async_copy(edges_hbm.at[pl.ds(baseA + 8, 8)], src_g.at[1],
                         sem_idx)

        def grp(gi, _):
            p = gi % 2
            for jj in range(8):
                pltpu.async_copy(ones_row, deg_target.at[src_g.at[p, jj]],
                                 sem_s, add=True)

            @pl.when(gi < AROWS // 8 - 1)
            def _():
                wait_group_load()

            for jj in range(8):
                wait_ones()

            @pl.when(gi < AROWS // 8 - 2)
            def _():
                pltpu.async_copy(
                    edges_hbm.at[pl.ds(baseA + (gi + 2) * 8, 8)],
                    src_g.at[p], sem_idx)
            return 0

        lax.fori_loop(0, AROWS // 8, grp, 0)

    with jax.named_scope("ph_degrees"):
        count_degrees(src_hbm, odeg_sh)
        count_degrees(dst_hbm, ideg_sh)
    plsc.subcore_barrier()

    stage = nrm_g.at[0, 0]
    for q in range(ROWS_PER_TILE // 128):
        pltpu.sync_copy(odeg_sh.at[pl.ds(r0 + q * 128, 128)], stage)
        for k in range(128 // L):
            x = jnp.maximum(nrm_g[0, 0, pl.ds(k * L, L)], 1.0)
            nrm_g[0, 0, pl.ds(k * L, L)] = _rsqrt16(x)
        pltpu.sync_copy(stage, onorm_sh.at[pl.ds(r0 + q * 128, 128)])

    @pl.when(c == 0)
    def _():
        pltpu.sync_copy(ideg_sh.at[pl.ds(r0, ROWS_PER_TILE)],
                        ideg_hbm.at[pl.ds(r0, ROWS_PER_TILE)])

    plsc.subcore_barrier()

    baseB = (c * NS + s) * BROWS

    def load_grp(g, slot):
        pltpu.async_copy(src_hbm.at[pl.ds(baseB + g * 8, 8)],
                         src_g.at[slot], sem_idx)
        pltpu.async_copy(dst_hbm.at[pl.ds(baseB + g * 8, 8)],
                         dst_g.at[slot], sem_idx)
        pltpu.async_copy(w_hbm.at[pl.ds(baseB + g * 8, 8)],
                         w_g.at[slot], sem_idx)

    def norm_gathers(slot):
        for jj in range(8):
            pltpu.async_copy(onorm_sh.at[src_g.at[slot, jj]],
                             nrm_g.at[slot, jj], sem_n)

    load_grp(0, 0)
    for _ in range(3):
        wait_group_load()
    norm_gathers(0)
    pltpu.async_copy(feat_hbm.at[src_g.at[0, 0]], rows3.at[0], sem_g)

    def chunk(j, _):
        g = j // 8
        jj = j % 8
        p = g % 2
        b = j % 2

        wait_gather()

        @pl.when(jj == 0)
        def _():
            for _ in range(8):
                wait_norm()

        def scale_body(gg, _):
            wn16 = (w_g[p, jj, pl.ds(gg * L, L)]
                    * nrm_g[p, jj, pl.ds(gg * L, L)])
            for quad in range(L // 4):
                vals = []
                for r in range(4):
                    i = gg * L + quad * 4 + r
                    sc = wn16[quad * 4 + r]
                    vals.extend(
                        rows3[b, i, pl.ds(k * L, L)] * sc
                        for k in range(D // L))
                for r in range(4):
                    i = gg * L + quad * 4 + r
                    for k in range(D // L):
                        rows3[b, i, pl.ds(k * L, L)] = vals[r * (D // L) + k]
            return 0

        def scale_range(lo, hi):
            @plsc.parallel_loop(lo, hi, 1, unroll=2)
            def _(gg):
                scale_body(gg, 0)

        scale_range(0, 4)

        @pl.when(j > 0)
        def _():
            wait_scatter()

        @pl.when(jnp.logical_and(jj == 0, g + 1 < NGRP))
        def _():
            load_grp(g + 1, 1 - p)

        @pl.when(jnp.logical_and(jj == 7, j < BROWS - 1))
        def _():
            for _ in range(3):
                wait_group_load()
            norm_gathers(1 - p)

        @pl.when(j < BROWS - 1)
        def _():
            g1 = (j + 1) // 8
            pltpu.async_copy(
                feat_hbm.at[src_g.at[g1 % 2, (j + 1) % 8]],
                rows3.at[1 - b], sem_g)

        scale_range(4, 8)
        pltpu.async_copy(rows3.at[b], agg_sh.at[dst_g.at[p, jj]], sem_s,
                         add=True)
        return 0

    with jax.named_scope("ph_aggregate"):
        lax.fori_loop(0, BROWS, chunk, 0)
    wait_scatter()
    plsc.subcore_barrier()

    pltpu.sync_copy(agg_sh.at[pl.ds(r0, ROWS_PER_TILE)],
                    part_hbm.at[c, pl.ds(r0, ROWS_PER_TILE)])


def _sc_aggregate(feat_p, ei2, w2):
    mesh = plsc.VectorSubcoreMesh(core_axis_name="c", subcore_axis_name="s",
                                  num_cores=NC, num_subcores=NS)
    return pl.kernel(
        _sc_body,
        out_type=(
            jax.ShapeDtypeStruct((NC, N_PAD, D), jnp.float32),
            jax.ShapeDtypeStruct((N_PAD,), jnp.float32),
        ),
        mesh=mesh,
        compiler_params=pltpu.CompilerParams(use_tc_tiling_on_sc=False),
        scratch_types=[
            pltpu.VMEM_SHARED((N_PAD, D), jnp.float32),
            pltpu.VMEM_SHARED((N_PAD,), jnp.float32),
            pltpu.VMEM_SHARED((N_PAD,), jnp.float32),
            pltpu.VMEM_SHARED((N_PAD,), jnp.float32),
            pltpu.VMEM((2, 8, 128), jnp.int32),
            pltpu.VMEM((2, 8, 128), jnp.int32),
            pltpu.VMEM((2, 8, 128), jnp.float32),
            pltpu.VMEM((2, 8, 128), jnp.float32),
            pltpu.VMEM((2, 128, D), jnp.float32),
            pltpu.SemaphoreType.DMA,
            pltpu.SemaphoreType.DMA,
            pltpu.SemaphoreType.DMA,
            pltpu.SemaphoreType.DMA,
        ],
        name="gcn_sc_aggregate",
    )(feat_p, ei2, w2)


ROWS_BLK = 2048
N_BLKS = N_PAD // ROWS_BLK


def _tc_tail_body(pa_ref, ideg_ref, b3_ref, w1_ref, bd1_ref, w2_ref, bd2_ref,
                  w3_ref, bd3_ref, out_ref, acc_ref):
    i = pl.program_id(0)

    @pl.when(i == 0)
    def _():
        acc_ref[...] = jnp.zeros_like(acc_ref)

    agg = pa_ref[0] + pa_ref[1]
    ideg = jnp.maximum(ideg_ref[...], 1.0)
    inorm = lax.rsqrt(ideg)
    h = jnp.maximum(agg * inorm + b3_ref[...], 0.0)
    row = i * ROWS_BLK + lax.broadcasted_iota(jnp.int32, (ROWS_BLK, 1), 0)
    h = jnp.where(row < N, h, 0.0)
    acc_ref[...] += jnp.sum(h, axis=0, keepdims=True)

    @pl.when(i == N_BLKS - 1)
    def _():
        hg = acc_ref[...] * (1.0 / N)
        m = jnp.maximum(jnp.dot(hg, w1_ref[...],
                                preferred_element_type=jnp.float32)
                        + bd1_ref[...], 0.0)
        m = jnp.maximum(jnp.dot(m, w2_ref[...],
                                preferred_element_type=jnp.float32)
                        + bd2_ref[...], 0.0)
        z = jnp.dot(m, w3_ref[...],
                    preferred_element_type=jnp.float32) + bd3_ref[...]
        out_ref[...] = 1.0 / (1.0 + jnp.exp(-z))


def _tc_tail(part, ideg2, b3, W1, bd1, W2, bd2, W3, bd3):
    full = lambda shape: pl.BlockSpec(shape, lambda i: tuple(0 for _ in shape))
    return pl.pallas_call(
        _tc_tail_body,
        grid=(N_BLKS,),
        in_specs=[
            pl.BlockSpec((NC, ROWS_BLK, D), lambda i: (0, i, 0)),
            pl.BlockSpec((ROWS_BLK, 1), lambda i: (i, 0)),
            full((1, D)),
            full((D, 16)), full((1, 16)),
            full((16, 8)), full((1, 8)),
            full((8, 1)), full((1, 1)),
        ],
        out_specs=pl.BlockSpec((1, 1), lambda i: (0, 0)),
        out_shape=jax.ShapeDtypeStruct((1, 1), jnp.float32),
        scratch_shapes=[pltpu.VMEM((1, D), jnp.float32)],
    )(part, ideg2, b3, W1, bd1, W2, bd2, W3, bd3)


def kernel(features, edge_index, e_weight, b1, b2, b3, W1, bd1, W2, bd2, W3,
           bd3):
    del b1, b2
    src = edge_index[0].astype(jnp.int32)
    dst = edge_index[1].astype(jnp.int32)
    w = e_weight.astype(jnp.float32)

    npad = E_PAD - E
    pad_idx = (N + (jnp.arange(npad, dtype=jnp.int32) % (N_PAD - N)))
    ei2 = jnp.concatenate(
        [jnp.stack([src, dst]), jnp.broadcast_to(pad_idx, (2, npad))],
        axis=1).reshape(2, EROWS, 128)
    w2 = jnp.concatenate([w, jnp.zeros((npad,), jnp.float32)]).reshape(
        EROWS, 128)
    feat_p = jnp.pad(features, ((0, N_PAD - N), (0, 0)))

    part, ideg = _sc_aggregate(feat_p, ei2, w2)
    return _tc_tail(part, ideg.reshape(N_PAD, 1), b3.reshape(1, D),
                    W1, bd1.reshape(1, 16), W2, bd2.reshape(1, 8),
                    W3, bd3.reshape(1, 1))
